# baseline (speedup 1.0000x reference)
import functools

import numpy as np
import jax
import jax.numpy as jnp
from jax import lax
from jax.experimental import pallas as pl
from jax.experimental.pallas import tpu as pltpu
from jax.experimental.pallas import tpu_sc as plsc

PRE_NMS_TOPK = 2000
MAX_OUT = 500
THR2 = np.float32(1.1 ** 2)
H = W = 128
BLK = 2048
PPAD = 20480


SLAB = 18 * 1536
ZOFF = SLAB


def _sc_im2col_kernel(src_hbm, out_hbm, slab_v, buf_v):
    wid = lax.axis_index("s") * 2 + lax.axis_index("c")
    pltpu.sync_copy(src_hbm.at[pl.ds(wid * 24576, SLAB)],
                    slab_v.at[pl.ds(0, SLAB)])
    zv = jnp.zeros((16,), jnp.float32)
    slab_v[pl.ds(ZOFF, 16)] = zv

    def clear(g, _):
        buf_v[pl.ds(g * 16, 16)] = zv
        return 0

    lax.fori_loop(0, 4096, clear, 0)
    ii = lax.iota(jnp.int32, 16)

    def plane(P, _):
        p, q = P // 2, P % 2

        def tap(t, _):
            i2l = t // 9
            di = (t % 9) // 3
            dj = t % 3
            rl = 4 * i2l + 2 * p + di
            for c in range(3):
                for g in range(8):
                    j2 = g * 16 + ii
                    jj = 4 * j2 + 2 * q + dj
                    gidx = jnp.where(jj >= 512,
                                     ZOFF + ii, rl * 1536 + jj * 3 + c)
                    v = plsc.load_gather(slab_v, [gidx])
                    sidx = (i2l * 128 + j2) * 128 + (di * 9 + dj * 3 + c)
                    plsc.store_scatter(buf_v, [sidx], v)
            return 0

        lax.fori_loop(0, 36, tap, 0)
        pltpu.sync_copy(
            buf_v, out_hbm.at[pl.ds(P * 2097152 + wid * 65536, 65536)])
        return 0

    lax.fori_loop(0, 4, plane, 0)


@functools.cache
def _sc_im2col():
    return pl.kernel(
        _sc_im2col_kernel,
        out_type=jax.ShapeDtypeStruct((8388608,), jnp.float32),
        mesh=plsc.VectorSubcoreMesh(core_axis_name="c", subcore_axis_name="s"),
        scratch_types=[pltpu.VMEM((SLAB + 16,), jnp.float32),
                       pltpu.VMEM((65536,), jnp.float32)],
        compiler_params=pltpu.CompilerParams(needs_layout_passes=False),
    )


def _conv1_kernel(p_ref, w_ref, o_ref):
    bb = pl.program_id(1)

    @pl.when(bb < 4)
    def _():
        o_ref[0] = jnp.maximum(
            jnp.dot(p_ref[...], w_ref[...],
                    preferred_element_type=jnp.float32), 0.0)

    @pl.when(bb >= 4)
    def _():
        o_ref[...] = jnp.zeros_like(o_ref)


def _conv2_head_kernel(f00, f01, f10, f11, w2_ref, w3_ref,
                       os_ref, oy_ref, ox_ref):
    base = pl.program_id(0) * BLK
    planes = ((f00, f01), (f10, f11))
    r_loc = lax.broadcasted_iota(jnp.int32, (BLK, 64), 0)
    jmask = (r_loc % W) != (W - 1)
    acc = jnp.zeros((BLK, 64), jnp.float32)
    for di in range(3):
        for dj in range(3):
            shift = (di >> 1) * W + (dj >> 1)
            sl = planes[di & 1][dj & 1][pl.ds(base + shift, BLK), :]
            if dj >> 1:
                sl = jnp.where(jmask, sl, 0.0)
            wtap = w2_ref[pl.ds(64 * (di * 3 + dj), 64), :]
            acc = acc + jnp.dot(sl, wtap, preferred_element_type=jnp.float32)
    h = jnp.maximum(acc, 0.0)
    g = jnp.dot(h, w3_ref[...], preferred_element_type=jnp.float32)

    rg = base + lax.broadcasted_iota(jnp.int32, (BLK, 1), 0)
    sval = 1.0 / (1.0 + jnp.exp(-g[:, 0:1]))
    yval = g[:, 1:2] + (jnp.floor_divide(rg, W).astype(jnp.float32) + 0.5)
    xval = g[:, 2:3] + (jnp.remainder(rg, W).astype(jnp.float32) + 0.5)

    r2 = lax.broadcasted_iota(jnp.int32, (BLK, W), 0)
    lane = lax.broadcasted_iota(jnp.int32, (BLK, W), 1)
    bsel = (r2 % W) == lane
    arow = lax.broadcasted_iota(jnp.int32, (BLK // W, BLK), 1)
    asel = (jnp.floor_divide(arow, W) ==
            lax.broadcasted_iota(jnp.int32, (BLK // W, BLK), 0)).astype(jnp.float32)
    for val, ref in ((sval, os_ref), (yval, oy_ref), (xval, ox_ref)):
        m = jnp.where(bsel, val, 0.0)
        ref[...] = jnp.dot(asel, m, preferred_element_type=jnp.float32,
                           precision=lax.Precision.HIGHEST)


def _nms_kernel(s_ref, y_ref, x_ref, o_ref, w_ref):
    s = s_ref[...]
    Y = y_ref[...]
    X = x_ref[...]
    sbits = lax.bitcast_convert_type(s, jnp.int32)

    def bs_body(_, carry):
        lo, hi = carry
        mid = (lo + hi) // 2
        c = jnp.sum((sbits >= mid).astype(jnp.int32))
        big = c >= PRE_NMS_TOPK
        return jnp.where(big, mid, lo), jnp.where(big, hi, mid)

    lo, _ = lax.fori_loop(0, 31, bs_body,
                          (jnp.int32(0), jnp.int32(0x3F800001)))

    cnt_gt = jnp.sum((sbits > lo).astype(jnp.int32))
    quota = (PRE_NMS_TOPK - cnt_gt).astype(jnp.float32)
    eqf = (sbits == lo).astype(jnp.float32)
    r_i = lax.broadcasted_iota(jnp.int32, (H, W), 0)
    c_i = lax.broadcasted_iota(jnp.int32, (H, W), 1)
    lt = (r_i <= c_i).astype(jnp.float32)
    sl = (c_i < r_i).astype(jnp.float32)
    ones = jnp.ones((H, W), jnp.float32)
    cum_in = jnp.dot(eqf, lt, preferred_element_type=jnp.float32,
                     precision=lax.Precision.HIGHEST)
    rsmat = jnp.dot(eqf, ones, preferred_element_type=jnp.float32,
                    precision=lax.Precision.HIGHEST)
    pr = jnp.dot(sl, rsmat, preferred_element_type=jnp.float32,
                 precision=lax.Precision.HIGHEST)
    pexcl = cum_in - eqf + pr
    elig = (sbits > lo) | ((sbits == lo) & (pexcl < quota))
    w_ref[...] = jnp.where(elig, s, -1.0)

    idx = r_i * W + c_i

    def nms_body(k, _):
        wv = w_ref[...]
        m = jnp.max(wv)
        cand = jnp.where(wv == m, idx, jnp.int32(1 << 30))
        isel = jnp.min(cand)
        oh = idx == isel
        ysel = jnp.sum(jnp.where(oh, Y, 0.0))
        xsel = jnp.sum(jnp.where(oh, X, 0.0))
        dy = Y - ysel
        dx = X - xsel
        d2 = dy * dy + dx * dx
        w_ref[...] = jnp.where(d2 <= THR2, -1.0, wv)
        valid = m > 0.0
        outs = jnp.where(valid, m, -1.0)
        outy = jnp.where(valid, ysel * 4.0, -1.0)
        outx = jnp.where(valid, xsel * 4.0, -1.0)
        lane = lax.broadcasted_iota(jnp.int32, (1, 128), 1)
        row = jnp.where(lane == 0, outs, jnp.where(lane == 1, outy, outx))
        o_ref[pl.ds(k, 1), :] = row
        return 0

    lax.fori_loop(0, MAX_OUT, nms_body, 0)


def kernel(image, W1, W2, Wscore, Wreg):
    xsrc = jnp.concatenate(
        [image.reshape(786432), jnp.zeros(24576, jnp.float32)])
    p1 = _sc_im2col()(xsrc).reshape(65536, 128)
    w1 = jnp.pad(W1.reshape(27, 64), ((0, 101), (0, 0)))

    f1p = pl.pallas_call(
        _conv1_kernel,
        grid=(4, 5),
        in_specs=[pl.BlockSpec((4096, 128),
                               lambda P, b: (jnp.minimum(P * 4 + b, 15), 0)),
                  pl.BlockSpec((128, 64), lambda P, b: (0, 0))],
        out_specs=pl.BlockSpec((1, 4096, 64), lambda P, b: (P, b, 0)),
        out_shape=jax.ShapeDtypeStruct((4, PPAD, 64), jnp.float32),
    )(p1, w1)

    fp = f1p
    w2 = W2.reshape(576, 64)
    w3 = jnp.pad(jnp.concatenate([Wscore[0, 0], Wreg[0, 0]], axis=1),
                 ((0, 0), (0, 125)))

    full = lambda shape: pl.BlockSpec(shape, lambda i: tuple(0 for _ in shape))
    s, yv, xv = pl.pallas_call(
        _conv2_head_kernel,
        grid=(16384 // BLK,),
        in_specs=[full((PPAD, 64))] * 4 + [full((576, 64)), full((64, 128))],
        out_specs=[pl.BlockSpec((BLK // W, W), lambda i: (i, 0))] * 3,
        out_shape=[jax.ShapeDtypeStruct((H, W), jnp.float32)] * 3,
    )(fp[0], fp[1], fp[2], fp[3], w2, w3)

    out = pl.pallas_call(
        _nms_kernel,
        out_shape=jax.ShapeDtypeStruct((512, 128), jnp.float32),
        scratch_shapes=[pltpu.VMEM((H, W), jnp.float32)],
    )(s, yv, xv)

    return out[:MAX_OUT, :3][None]


# speedup vs baseline: 1.0501x; 1.0501x over previous
import functools

import numpy as np
import jax
import jax.numpy as jnp
from jax import lax
from jax.experimental import pallas as pl
from jax.experimental.pallas import tpu as pltpu
from jax.experimental.pallas import tpu_sc as plsc

PRE_NMS_TOPK = 2000
MAX_OUT = 500
THR2 = np.float32(1.1 ** 2)
H = W = 128
BLK = 2048
PPAD = 16520


SLAB = 18 * 1536
ZOFF = SLAB


def _sc_im2col_kernel(src_hbm, out_hbm, slab_v, buf_v):
    wid = lax.axis_index("s") * 2 + lax.axis_index("c")
    pltpu.sync_copy(src_hbm.at[pl.ds(wid * 24576, SLAB)],
                    slab_v.at[pl.ds(0, SLAB)])
    zv = jnp.zeros((16,), jnp.float32)
    slab_v[pl.ds(ZOFF, 16)] = zv

    def clear(g, _):
        for u in range(16):
            buf_v[pl.ds(g * 256 + u * 16, 16)] = zv
        return 0

    lax.fori_loop(0, 256, clear, 0)
    ii = lax.iota(jnp.int32, 16)

    def plane(P, _):
        p, q = P // 2, P % 2

        def tap(t, _):
            i2l = t // 9
            di = (t % 9) // 3
            dj = t % 3
            rl = 4 * i2l + 2 * p + di
            for c in range(3):
                for g in range(8):
                    j2 = g * 16 + ii
                    jj = 4 * j2 + 2 * q + dj
                    gidx = jnp.where(jj >= 512,
                                     ZOFF + ii, rl * 1536 + jj * 3 + c)
                    v = plsc.load_gather(slab_v, [gidx])
                    sidx = (i2l * 128 + j2) * 128 + (di * 9 + dj * 3 + c)
                    plsc.store_scatter(buf_v, [sidx], v)
            return 0

        lax.fori_loop(0, 36, tap, 0)
        pltpu.sync_copy(
            buf_v, out_hbm.at[pl.ds(P * 2097152 + wid * 65536, 65536)])
        return 0

    lax.fori_loop(0, 4, plane, 0)


@functools.cache
def _sc_im2col():
    return pl.kernel(
        _sc_im2col_kernel,
        out_type=jax.ShapeDtypeStruct((8388608,), jnp.float32),
        mesh=plsc.VectorSubcoreMesh(core_axis_name="c", subcore_axis_name="s"),
        scratch_types=[pltpu.VMEM((SLAB + 16,), jnp.float32),
                       pltpu.VMEM((65536,), jnp.float32)],
        compiler_params=pltpu.CompilerParams(needs_layout_passes=False),
    )


def _conv1_kernel(p_ref, w_ref, o_ref):
    o_ref[...] = jnp.maximum(
        jnp.dot(p_ref[...], w_ref[...], preferred_element_type=jnp.float32), 0.0)


def _conv2_head_kernel(f00, f01, f10, f11, w2_ref, w3_ref,
                       os_ref, oy_ref, ox_ref):
    base = pl.program_id(0) * BLK
    planes = ((f00, f01), (f10, f11))
    r_loc = lax.broadcasted_iota(jnp.int32, (BLK, 64), 0)
    jmask = (r_loc % W) != (W - 1)
    acc = jnp.zeros((BLK, 64), jnp.float32)
    for di in range(3):
        for dj in range(3):
            shift = (di >> 1) * W + (dj >> 1)
            sl = planes[di & 1][dj & 1][pl.ds(base + shift, BLK), :]
            if dj >> 1:
                sl = jnp.where(jmask, sl, 0.0)
            wtap = w2_ref[pl.ds(64 * (di * 3 + dj), 64), :]
            acc = acc + jnp.dot(sl, wtap, preferred_element_type=jnp.float32)
    h = jnp.maximum(acc, 0.0)
    g = jnp.dot(h, w3_ref[...], preferred_element_type=jnp.float32)

    rg = base + lax.broadcasted_iota(jnp.int32, (BLK, 1), 0)
    sval = 1.0 / (1.0 + jnp.exp(-g[:, 0:1]))
    yval = g[:, 1:2] + (jnp.floor_divide(rg, W).astype(jnp.float32) + 0.5)
    xval = g[:, 2:3] + (jnp.remainder(rg, W).astype(jnp.float32) + 0.5)

    r2 = lax.broadcasted_iota(jnp.int32, (BLK, W), 0)
    lane = lax.broadcasted_iota(jnp.int32, (BLK, W), 1)
    bsel = (r2 % W) == lane
    arow = lax.broadcasted_iota(jnp.int32, (BLK // W, BLK), 1)
    asel = (jnp.floor_divide(arow, W) ==
            lax.broadcasted_iota(jnp.int32, (BLK // W, BLK), 0)).astype(jnp.float32)
    for val, ref in ((sval, os_ref), (yval, oy_ref), (xval, ox_ref)):
        m = jnp.where(bsel, val, 0.0)
        ref[...] = jnp.dot(asel, m, preferred_element_type=jnp.float32,
                           precision=lax.Precision.HIGHEST)


def _nms_kernel(s_ref, y_ref, x_ref, o_ref, w_ref):
    s = s_ref[...]
    Y = y_ref[...]
    X = x_ref[...]
    sbits = lax.bitcast_convert_type(s, jnp.int32)

    def bs_body(_, carry):
        lo, hi = carry
        mid = (lo + hi) // 2
        c = jnp.sum((sbits >= mid).astype(jnp.int32))
        big = c >= PRE_NMS_TOPK
        return jnp.where(big, mid, lo), jnp.where(big, hi, mid)

    lo, _ = lax.fori_loop(0, 31, bs_body,
                          (jnp.int32(0), jnp.int32(0x3F800001)))

    cnt_gt = jnp.sum((sbits > lo).astype(jnp.int32))
    quota = (PRE_NMS_TOPK - cnt_gt).astype(jnp.float32)
    eqf = (sbits == lo).astype(jnp.float32)
    r_i = lax.broadcasted_iota(jnp.int32, (H, W), 0)
    c_i = lax.broadcasted_iota(jnp.int32, (H, W), 1)
    lt = (r_i <= c_i).astype(jnp.float32)
    sl = (c_i < r_i).astype(jnp.float32)
    ones = jnp.ones((H, W), jnp.float32)
    cum_in = jnp.dot(eqf, lt, preferred_element_type=jnp.float32,
                     precision=lax.Precision.HIGHEST)
    rsmat = jnp.dot(eqf, ones, preferred_element_type=jnp.float32,
                    precision=lax.Precision.HIGHEST)
    pr = jnp.dot(sl, rsmat, preferred_element_type=jnp.float32,
                 precision=lax.Precision.HIGHEST)
    pexcl = cum_in - eqf + pr
    elig = (sbits > lo) | ((sbits == lo) & (pexcl < quota))
    w_ref[...] = jnp.where(elig, s, -1.0)

    idx = r_i * W + c_i

    def nms_body(k, _):
        wv = w_ref[...]
        m = jnp.max(wv)
        cand = jnp.where(wv == m, idx, jnp.int32(1 << 30))
        isel = jnp.min(cand)
        oh = idx == isel
        ysel = jnp.sum(jnp.where(oh, Y, 0.0))
        xsel = jnp.sum(jnp.where(oh, X, 0.0))
        dy = Y - ysel
        dx = X - xsel
        d2 = dy * dy + dx * dx
        w_ref[...] = jnp.where(d2 <= THR2, -1.0, wv)
        valid = m > 0.0
        outs = jnp.where(valid, m, -1.0)
        outy = jnp.where(valid, ysel * 4.0, -1.0)
        outx = jnp.where(valid, xsel * 4.0, -1.0)
        lane = lax.broadcasted_iota(jnp.int32, (1, 128), 1)
        row = jnp.where(lane == 0, outs, jnp.where(lane == 1, outy, outx))
        o_ref[pl.ds(k, 1), :] = row
        return 0

    lax.fori_loop(0, MAX_OUT, nms_body, 0)


def kernel(image, W1, W2, Wscore, Wreg):
    xsrc = jnp.concatenate(
        [image.reshape(786432), jnp.zeros(24576, jnp.float32)])
    p1 = _sc_im2col()(xsrc).reshape(65536, 128)
    w1 = jnp.pad(W1.reshape(27, 64), ((0, 101), (0, 0)))

    f1p = pl.pallas_call(
        _conv1_kernel,
        grid=(16,),
        in_specs=[pl.BlockSpec((4096, 128), lambda i: (i, 0)),
                  pl.BlockSpec((128, 64), lambda i: (0, 0))],
        out_specs=pl.BlockSpec((4096, 64), lambda i: (i, 0)),
        out_shape=jax.ShapeDtypeStruct((65536, 64), jnp.float32),
    )(p1, w1)

    fp = jnp.pad(f1p.reshape(4, 16384, 64),
                 ((0, 0), (0, PPAD - 16384), (0, 0)))
    w2 = W2.reshape(576, 64)
    w3 = jnp.pad(jnp.concatenate([Wscore[0, 0], Wreg[0, 0]], axis=1),
                 ((0, 0), (0, 125)))

    full = lambda shape: pl.BlockSpec(shape, lambda i: tuple(0 for _ in shape))
    s, yv, xv = pl.pallas_call(
        _conv2_head_kernel,
        grid=(16384 // BLK,),
        in_specs=[full((PPAD, 64))] * 4 + [full((576, 64)), full((64, 128))],
        out_specs=[pl.BlockSpec((BLK // W, W), lambda i: (i, 0))] * 3,
        out_shape=[jax.ShapeDtypeStruct((H, W), jnp.float32)] * 3,
    )(fp[0], fp[1], fp[2], fp[3], w2, w3)

    out = pl.pallas_call(
        _nms_kernel,
        out_shape=jax.ShapeDtypeStruct((512, 128), jnp.float32),
        scratch_shapes=[pltpu.VMEM((H, W), jnp.float32)],
    )(s, yv, xv)

    return out[:MAX_OUT, :3][None]
